# Initial kernel scaffold; baseline (speedup 1.0000x reference)
import jax
import jax.numpy as jnp
from jax.experimental import pallas as pl


def kernel(x, W, att_src, att_dst, bias, gamma, beta):
    raise NotImplementedError("write your pallas kernel here")


# speedup vs baseline: 34.0693x; 34.0693x over previous
import functools

import jax
import jax.numpy as jnp
from jax.experimental import pallas as pl
from jax.experimental.pallas import tpu as pltpu

B, T, H = 2, 2048, 768
HEADS = 4
LAYERS = 3
WINDOW = 2
NEG = 0.2
EPS = 1e-5
NEG_BIG = -1e30


def _shift_rows(arr, k):
    if k == 0:
        return arr
    n = arr.shape[0]
    z = jnp.zeros((abs(k),) + arr.shape[1:], arr.dtype)
    if k > 0:
        return jnp.concatenate([arr[k:], z], axis=0)
    return jnp.concatenate([z, arr[: n + k]], axis=0)


def _layer_body(x_ref, w_ref, asrc_ref, adst_ref, bias_ref, gamma_ref,
                beta_ref, out_ref, acc_ref):
    hd = pl.program_id(1)
    xb = x_ref[0]
    h = jnp.dot(xb, w_ref[0], preferred_element_type=jnp.float32)

    a_s = jnp.sum(h * asrc_ref[0], axis=1, keepdims=True)
    a_d = jnp.sum(h * adst_ref[0], axis=1, keepdims=True)

    t_idx = jax.lax.broadcasted_iota(jnp.int32, (T, 1), 0)
    offs = range(-WINDOW, WINDOW + 1)
    es = []
    for k in offs:
        valid = (t_idx + k >= 0) & (t_idx + k < T)
        e = _shift_rows(a_s, k) + a_d
        e = jnp.where(e > 0, e, NEG * e)
        es.append((jnp.where(valid, e, NEG_BIG), valid))
    m = functools.reduce(jnp.maximum, [e for e, _ in es])
    exs = [jnp.where(v, jnp.exp(e - m), 0.0) for e, v in es]
    den = functools.reduce(jnp.add, exs) + 1e-16

    out_h = None
    for k, ex in zip(offs, exs):
        contrib = (ex / den) * _shift_rows(h, k)
        out_h = contrib if out_h is None else out_h + contrib

    @pl.when(hd == 0)
    def _():
        acc_ref[...] = out_h

    @pl.when(hd != 0)
    def _():
        acc_ref[...] = acc_ref[...] + out_h

    @pl.when(hd == HEADS - 1)
    def _():
        z = acc_ref[...] * (1.0 / HEADS) + bias_ref[...] + xb
        mu = jnp.mean(z, axis=1, keepdims=True)
        var = jnp.mean((z - mu) ** 2, axis=1, keepdims=True)
        y = (z - mu) * jax.lax.rsqrt(var + EPS) * gamma_ref[...] + beta_ref[...]
        out_ref[0] = y


def _gat_layer(nf, w_l, asrc_l, adst_l, bias_l, gamma_l, beta_l, interpret=False):
    return pl.pallas_call(
        _layer_body,
        grid=(B, HEADS),
        in_specs=[
            pl.BlockSpec((1, T, H), lambda b, h: (b, 0, 0)),
            pl.BlockSpec((1, H, H), lambda b, h: (h, 0, 0)),
            pl.BlockSpec((1, 1, H), lambda b, h: (h, 0, 0)),
            pl.BlockSpec((1, 1, H), lambda b, h: (h, 0, 0)),
            pl.BlockSpec((1, H), lambda b, h: (0, 0)),
            pl.BlockSpec((1, H), lambda b, h: (0, 0)),
            pl.BlockSpec((1, H), lambda b, h: (0, 0)),
        ],
        out_specs=pl.BlockSpec((1, T, H), lambda b, h: (b, 0, 0)),
        out_shape=jax.ShapeDtypeStruct((B, T, H), jnp.float32),
        scratch_shapes=[pltpu.VMEM((T, H), jnp.float32)],
        interpret=interpret,
    )(nf, w_l, asrc_l, adst_l, bias_l, gamma_l, beta_l)


def kernel(x, W, att_src, att_dst, bias, gamma, beta):
    Wr = W.reshape(LAYERS, H, HEADS, H).transpose(0, 2, 1, 3)
    asrc = att_src.reshape(LAYERS, HEADS, 1, H)
    adst = att_dst.reshape(LAYERS, HEADS, 1, H)
    nf = x
    for l in range(LAYERS):
        nf = _gat_layer(nf, Wr[l], asrc[l], adst[l],
                        bias[l].reshape(1, H), gamma[l].reshape(1, H),
                        beta[l].reshape(1, H))
    return nf


# speedup vs baseline: 37.6712x; 1.1057x over previous
import functools

import jax
import jax.numpy as jnp
from jax.experimental import pallas as pl
from jax.experimental.pallas import tpu as pltpu

B, T, H = 2, 2048, 768
HEADS = 4
LAYERS = 3
WINDOW = 2
NEG = 0.2
EPS = 1e-5
NEG_BIG = -1e30


def _shift_rows(arr, k):
    if k == 0:
        return arr
    n = arr.shape[0]
    z = jnp.zeros((abs(k),) + arr.shape[1:], arr.dtype)
    if k > 0:
        return jnp.concatenate([arr[k:], z], axis=0)
    return jnp.concatenate([z, arr[: n + k]], axis=0)


def _layer_body(x_ref, w_ref, asrc_ref, adst_ref, bias_ref, gamma_ref,
                beta_ref, out_ref, acc_ref):
    hd = pl.program_id(1)
    xb = x_ref[0]
    h = jnp.dot(xb.astype(jnp.bfloat16), w_ref[0],
                preferred_element_type=jnp.float32)

    a_s = jnp.sum(h * asrc_ref[0], axis=1, keepdims=True)
    a_d = jnp.sum(h * adst_ref[0], axis=1, keepdims=True)

    t_idx = jax.lax.broadcasted_iota(jnp.int32, (T, 1), 0)
    offs = range(-WINDOW, WINDOW + 1)
    es = []
    for k in offs:
        valid = (t_idx + k >= 0) & (t_idx + k < T)
        e = _shift_rows(a_s, k) + a_d
        e = jnp.where(e > 0, e, NEG * e)
        es.append((jnp.where(valid, e, NEG_BIG), valid))
    m = functools.reduce(jnp.maximum, [e for e, _ in es])
    exs = [jnp.where(v, jnp.exp(e - m), 0.0) for e, v in es]
    den = functools.reduce(jnp.add, exs) + 1e-16

    out_h = None
    for k, ex in zip(offs, exs):
        contrib = (ex / den) * _shift_rows(h, k)
        out_h = contrib if out_h is None else out_h + contrib

    @pl.when(hd == 0)
    def _():
        acc_ref[...] = out_h

    @pl.when(hd != 0)
    def _():
        acc_ref[...] = acc_ref[...] + out_h

    @pl.when(hd == HEADS - 1)
    def _():
        z = acc_ref[...] * (1.0 / HEADS) + bias_ref[...] + xb
        mu = jnp.mean(z, axis=1, keepdims=True)
        var = jnp.mean((z - mu) ** 2, axis=1, keepdims=True)
        y = (z - mu) * jax.lax.rsqrt(var + EPS) * gamma_ref[...] + beta_ref[...]
        out_ref[0] = y


def _gat_layer(nf, w_l, asrc_l, adst_l, bias_l, gamma_l, beta_l, interpret=False):
    return pl.pallas_call(
        _layer_body,
        grid=(B, HEADS),
        in_specs=[
            pl.BlockSpec((1, T, H), lambda b, h: (b, 0, 0)),
            pl.BlockSpec((1, H, H), lambda b, h: (h, 0, 0)),
            pl.BlockSpec((1, 1, H), lambda b, h: (h, 0, 0)),
            pl.BlockSpec((1, 1, H), lambda b, h: (h, 0, 0)),
            pl.BlockSpec((1, H), lambda b, h: (0, 0)),
            pl.BlockSpec((1, H), lambda b, h: (0, 0)),
            pl.BlockSpec((1, H), lambda b, h: (0, 0)),
        ],
        out_specs=pl.BlockSpec((1, T, H), lambda b, h: (b, 0, 0)),
        out_shape=jax.ShapeDtypeStruct((B, T, H), jnp.float32),
        scratch_shapes=[pltpu.VMEM((T, H), jnp.float32)],
        interpret=interpret,
    )(nf, w_l, asrc_l, adst_l, bias_l, gamma_l, beta_l)


def kernel(x, W, att_src, att_dst, bias, gamma, beta):
    Wr = W.reshape(LAYERS, H, HEADS, H).transpose(0, 2, 1, 3).astype(jnp.bfloat16)
    asrc = att_src.reshape(LAYERS, HEADS, 1, H)
    adst = att_dst.reshape(LAYERS, HEADS, 1, H)
    nf = x
    for l in range(LAYERS):
        nf = _gat_layer(nf, Wr[l], asrc[l], adst[l],
                        bias[l].reshape(1, H), gamma[l].reshape(1, H),
                        beta[l].reshape(1, H))
    return nf
